# baseline (speedup 1.0000x reference)
import functools

import jax
import jax.numpy as jnp
from jax import lax
from jax.experimental import pallas as pl
from jax.experimental.pallas import tpu as pltpu
from jax.experimental.pallas import tpu_sc as plsc

B = 4096
H = 64
NC = 2
NS = 16
NW = NC * NS
C = 512

N_STATIC = B * 8
N_HIST = B * 200
N_FUT = B * 50

_MESH = dict(core_axis_name="c", subcore_axis_name="s",
             num_cores=NC, num_subcores=NS)


def _make_pipeline(idx_v, rows_v, sems_g, sems_w, w):
    def run_stream(table, off, base0, per_w, dst_fn):
        m = per_w // C
        tail = per_w - m * C

        def start_chunk(buf, lbase, n):
            for j in range(n // 128):
                pltpu.async_copy(
                    table.at[idx_v.at[pl.ds(off + lbase + j * 128, 128)]],
                    rows_v.at[buf, pl.ds(j * 128, 128)], sems_g[buf])

        def wait_chunk(buf, n):
            pltpu.make_async_copy(table.at[pl.ds(0, n)],
                                  rows_v.at[buf, pl.ds(0, n)],
                                  sems_g[buf]).wait()

        def start_write(buf, base, n):
            pltpu.async_copy(rows_v.at[buf, pl.ds(0, n)], dst_fn(base, n),
                             sems_w[buf])

        def wait_write(buf, base, n):
            pltpu.make_async_copy(rows_v.at[buf, pl.ds(0, n)],
                                  dst_fn(base, n), sems_w[buf]).wait()

        def bofs(i):
            return base0 + i * C

        start_chunk(0, 0, C)
        start_chunk(1, C, C)
        wait_chunk(0, C)
        start_write(0, bofs(0), C)

        def pair(k, _):
            i0 = 2 * k
            wait_write(0, bofs(i0 - 2), C)
            start_chunk(0, i0 * C, C)
            wait_chunk(1, C)
            start_write(1, bofs(i0 - 1), C)

            wait_write(1, bofs(i0 - 1), C)
            start_chunk(1, (i0 + 1) * C, C)
            wait_chunk(0, C)
            start_write(0, bofs(i0), C)
            return 0

        lax.fori_loop(1, m // 2, pair, 0)

        wait_chunk(1, C)
        start_write(1, bofs(m - 1), C)
        if tail:
            wait_write(0, bofs(m - 2), C)
            start_chunk(0, m * C, tail)
            wait_chunk(0, tail)
            start_write(0, bofs(m), tail)
            wait_write(0, bofs(m), tail)
        else:
            wait_write(0, bofs(m - 2), C)
        wait_write(1, bofs(m - 1), C)

    return run_stream


def _preload_idx(idx_refs, per_ws, offs, idx_v, si, w):
    for idx_hbm, per_w, off in zip(idx_refs, per_ws, offs):
        pltpu.async_copy(idx_hbm.at[pl.ds(w * per_w, per_w)],
                         idx_v.at[pl.ds(off, per_w)], si)
    for idx_hbm, per_w, off in zip(idx_refs, per_ws, offs):
        pltpu.make_async_copy(idx_hbm.at[pl.ds(0, per_w)],
                              idx_v.at[pl.ds(off, per_w)], si).wait()


def _interleave(out, parity):
    return lambda base, n: out.at[pl.ds(base, n), parity]


def _linear(out):
    return lambda base, n: out.at[pl.ds(base, n)]


PW_A = (N_STATIC // NW, N_STATIC // NW, N_FUT // NW)
OFF_A = (0, PW_A[0], PW_A[0] + PW_A[1])
IDX_A = sum(PW_A)


def _body_a(sc_idx, sca_idx, fu_idx, w_sc, w_sca, w_fu,
            out_s, out_f,
            idx_v, rows_v, sg0, sg1, sw0, sw1, si):
    w = lax.axis_index("s") * NC + lax.axis_index("c")
    _preload_idx((sc_idx, sca_idx, fu_idx), PW_A, OFF_A, idx_v, si, w)
    run = _make_pipeline(idx_v, rows_v, (sg0, sg1), (sw0, sw1), w)
    run(w_sc, OFF_A[0], w * PW_A[0], PW_A[0], _interleave(out_s, 0))
    run(w_sca, OFF_A[1], w * PW_A[1], PW_A[1], _interleave(out_s, 1))
    run(w_fu, OFF_A[2], w * PW_A[2], PW_A[2], _linear(out_f))


PW_B = (N_HIST // NW, N_HIST // NW)
OFF_B = (0, PW_B[0])
IDX_B = sum(PW_B)


CB = 256
NBUF = 4


def _body_b(hc_idx, hca_idx, w_hc, w_hca,
            out_h,
            idx_v, rows_v, sg0, sg1, sg2, sg3, sw0, sw1, sw2, sw3, si):
    w = lax.axis_index("s") * NC + lax.axis_index("c")
    _preload_idx((hc_idx, hca_idx), PW_B, OFF_B, idx_v, si, w)
    sems_g = (sg0, sg1, sg2, sg3)
    sems_w = (sw0, sw1, sw2, sw3)

    def run(table, off, base0, per_w, dst_fn):
        m = per_w // CB

        def start_chunk(buf, i):
            for j in range(CB // 128):
                pltpu.async_copy(
                    table.at[idx_v.at[pl.ds(off + i * CB + j * 128, 128)]],
                    rows_v.at[buf, pl.ds(j * 128, 128)], sems_g[buf])

        def wait_chunk(buf):
            pltpu.make_async_copy(table.at[pl.ds(0, CB)],
                                  rows_v.at[buf], sems_g[buf]).wait()

        def start_write(buf, i):
            pltpu.async_copy(rows_v.at[buf], dst_fn(base0 + i * CB, CB),
                             sems_w[buf])

        def wait_write(buf, i):
            pltpu.make_async_copy(rows_v.at[buf],
                                  dst_fn(base0 + i * CB, CB),
                                  sems_w[buf]).wait()

        for i in range(NBUF):
            start_chunk(i, i)
        wait_chunk(0)
        start_write(0, 0)
        wait_chunk(1)
        start_write(1, 1)

        def quad(k, _):
            i0 = NBUF * k
            for d in range(NBUF):
                i = i0 + d
                wait_write(d, i - NBUF)
                start_chunk(d, i)
                wait_chunk((d + 2) % NBUF)
                start_write((d + 2) % NBUF, i - 2)
            return 0

        lax.fori_loop(1, m // NBUF, quad, 0)

        wait_chunk((m - 2) % NBUF)
        start_write((m - 2) % NBUF, m - 2)
        wait_chunk((m - 1) % NBUF)
        start_write((m - 1) % NBUF, m - 1)
        for d in range(NBUF):
            wait_write((m - NBUF + d) % NBUF, m - NBUF + d)

    run(w_hc, OFF_B[0], w * PW_B[0], PW_B[0], _interleave(out_h, 0))
    run(w_hca, OFF_B[1], w * PW_B[1], PW_B[1], _interleave(out_h, 1))


def _sc_call(body, out_type, idx_words, nbuf, chunk):
    return pl.kernel(
        body,
        out_type=out_type,
        mesh=plsc.VectorSubcoreMesh(**_MESH),
        compiler_params=pltpu.CompilerParams(use_tc_tiling_on_sc=False),
        scratch_types=(
            [pltpu.VMEM((idx_words,), jnp.int32),
             pltpu.VMEM((nbuf, chunk, H), jnp.float32)]
            + [pltpu.SemaphoreType.DMA] * (2 * nbuf + 1)
        ),
    )


@jax.jit
def _embed(sc_idx, sca_idx, hc_idx, hca_idx, fu_idx,
           w_sc, w_sca, w_hc, w_hca, w_fu):
    out_h = _sc_call(_body_b, [
        jax.ShapeDtypeStruct((N_HIST, 2, H), jnp.float32),
    ], IDX_B, NBUF, CB)(hc_idx, hca_idx, w_hc, w_hca)[0]
    out_s, out_f = _sc_call(_body_a, [
        jax.ShapeDtypeStruct((N_STATIC, 2, H), jnp.float32),
        jax.ShapeDtypeStruct((N_FUT, H), jnp.float32),
    ], IDX_A, 2, C)(sc_idx, sca_idx, fu_idx, w_sc, w_sca, w_fu)
    return out_s, out_h, out_f


def kernel(static_cont_input, static_cat_input, history_cont_input,
           history_cat_input, future_input, W_static_cont, W_static_cat,
           W_history_cont, W_history_cat, W_future):
    def prep(idx):
        return idx.astype(jnp.int32).reshape(-1)

    out_s, out_h, out_f = _embed(
        prep(static_cont_input), prep(static_cat_input),
        prep(history_cont_input), prep(history_cat_input),
        prep(future_input),
        W_static_cont, W_static_cat, W_history_cont, W_history_cat, W_future)
    return (out_s.reshape(B, 8, 2 * H),
            out_h.reshape(B, 200, 2 * H),
            out_f.reshape(B, 50, H))
